# baseline (speedup 1.0000x reference)
import functools
import jax
import jax.numpy as jnp
from jax import lax
from jax.experimental import pallas as pl
from jax.experimental.pallas import tpu as pltpu
from jax.experimental.pallas import tpu_sc as plsc

NUM_TOKENS = 100000
HIDDEN_SIZE = 128
NUM_CORES = 2
NUM_SUBCORES = 16
NUM_WORKERS = NUM_CORES * NUM_SUBCORES
MAIN_ROWS = 99840
ROWS_PER_WORKER = MAIN_ROWS // NUM_WORKERS
CH = 312
NCH = ROWS_PER_WORKER // CH
TAIL_ROWS = NUM_TOKENS - MAIN_ROWS


def kernel(embedding_weight):
    mesh = plsc.VectorSubcoreMesh(core_axis_name="c", subcore_axis_name="s")

    @functools.partial(
        pl.kernel,
        mesh=mesh,
        out_type=jax.ShapeDtypeStruct((1, NUM_TOKENS, HIDDEN_SIZE), jnp.float32),
        scratch_types=[
            pltpu.VMEM((2, CH, HIDDEN_SIZE), jnp.float32),
            pltpu.SemaphoreType.DMA((2,)),
            pltpu.SemaphoreType.DMA((2,)),
        ],
    )
    def sc_copy(in_hbm, out_hbm, buf, isems, osems):
        wid = lax.axis_index("s") * NUM_CORES + lax.axis_index("c")
        base = wid * ROWS_PER_WORKER

        def in_cp(c):
            b = c % 2
            return pltpu.async_copy(
                in_hbm.at[pl.ds(base + c * CH, CH), :], buf.at[b], isems.at[b])

        def out_cp(c):
            b = c % 2
            return pltpu.async_copy(
                buf.at[b], out_hbm.at[0, pl.ds(base + c * CH, CH), :], osems.at[b])

        handles = [None] * NCH
        out_handles = [None] * NCH
        handles[0] = in_cp(0)
        for c in range(NCH):
            if c + 1 < NCH:
                if c >= 1:
                    out_handles[c - 1].wait()
                handles[c + 1] = in_cp(c + 1)
            handles[c].wait()
            out_handles[c] = out_cp(c)
        out_handles[NCH - 2].wait()
        out_handles[NCH - 1].wait()

        @pl.when(wid < TAIL_ROWS // 8)
        def _tail():
            tbase = MAIN_ROWS + wid * 8
            pltpu.sync_copy(
                in_hbm.at[pl.ds(tbase, 8), :], buf.at[0, pl.ds(0, 8), :])
            pltpu.sync_copy(
                buf.at[0, pl.ds(0, 8), :], out_hbm.at[0, pl.ds(tbase, 8), :])

    return sc_copy(embedding_weight)


# speedup vs baseline: 1.7420x; 1.7420x over previous
import jax
import jax.numpy as jnp
from jax.experimental import pallas as pl

NUM_TOKENS = 100000
HIDDEN_SIZE = 128
ROWS_PER_BLOCK = 20000


def _copy_block(in_ref, out_ref):
    out_ref[0] = in_ref[...]


def kernel(embedding_weight):
    grid = (NUM_TOKENS // ROWS_PER_BLOCK,)
    out = pl.pallas_call(
        _copy_block,
        grid=grid,
        in_specs=[
            pl.BlockSpec((ROWS_PER_BLOCK, HIDDEN_SIZE), lambda i: (i, 0)),
        ],
        out_specs=pl.BlockSpec((1, ROWS_PER_BLOCK, HIDDEN_SIZE), lambda i: (0, i, 0)),
        out_shape=jax.ShapeDtypeStruct((1, NUM_TOKENS, HIDDEN_SIZE), jnp.float32),
    )(embedding_weight)
    return out
